# baseline (speedup 1.0000x reference)
import jax
import jax.numpy as jnp
from jax.experimental import pallas as pl

NUM_EXPERTS = 16
TOP_K = 2
INPUT_DIM = 768
OUTPUT_DIM = 256
TOKEN_BLOCK = 256


def _moe_block_kernel(x_ref, wg_ref, bg_ref, we_ref, be_ref, o_ref):
    x = x_ref[...]
    logits = jax.lax.dot_general(
        x, wg_ref[...], (((1,), (1,)), ((), ())),
        preferred_element_type=jnp.float32) + bg_ref[...]
    w = jax.nn.softmax(logits, axis=-1)
    e_iota = jax.lax.broadcasted_iota(jnp.int32, w.shape, 1)
    i1 = jnp.argmax(w, axis=-1)[:, None]
    v1 = jnp.max(w, axis=-1)[:, None]
    w2 = jnp.where(e_iota == i1, -jnp.inf, w)
    i2 = jnp.argmax(w2, axis=-1)[:, None]
    v2 = jnp.max(w2, axis=-1)[:, None]
    cw = (jnp.where(e_iota == i1, v1, 0.0)
          + jnp.where(e_iota == i2, v2, 0.0))

    xb = x.astype(jnp.bfloat16)
    w2d = we_ref[...].reshape(
        NUM_EXPERTS * OUTPUT_DIM, INPUT_DIM).astype(jnp.bfloat16)
    y_all = jax.lax.dot_general(
        xb, w2d, (((1,), (1,)), ((), ())),
        preferred_element_type=jnp.float32)
    acc = jnp.zeros((x.shape[0], OUTPUT_DIM), jnp.float32)
    for e in range(NUM_EXPERTS):
        ye = y_all[:, e * OUTPUT_DIM:(e + 1) * OUTPUT_DIM]
        acc = acc + cw[:, e][:, None] * (ye + be_ref[e][None, :])
    o_ref[...] = acc


@jax.jit
def kernel(x, Wg, bg, We, be):
    bs, L, d = x.shape
    n_tokens = bs * L
    xf = x.reshape(n_tokens, d)
    grid = (n_tokens // TOKEN_BLOCK,)
    out = pl.pallas_call(
        _moe_block_kernel,
        grid=grid,
        in_specs=[
            pl.BlockSpec((TOKEN_BLOCK, d), lambda i: (i, 0)),
            pl.BlockSpec((NUM_EXPERTS, d), lambda i: (0, 0)),
            pl.BlockSpec((1, NUM_EXPERTS), lambda i: (0, 0)),
            pl.BlockSpec((NUM_EXPERTS, OUTPUT_DIM, d), lambda i: (0, 0, 0)),
            pl.BlockSpec((NUM_EXPERTS, OUTPUT_DIM), lambda i: (0, 0)),
        ],
        out_specs=pl.BlockSpec((TOKEN_BLOCK, OUTPUT_DIM), lambda i: (i, 0)),
        out_shape=jax.ShapeDtypeStruct((n_tokens, OUTPUT_DIM), jnp.float32),
    )(xf, Wg, bg.reshape(1, NUM_EXPERTS), We, be)
    return out.reshape(bs, L, OUTPUT_DIM)


# speedup vs baseline: 1.1033x; 1.1033x over previous
import jax
import jax.numpy as jnp
from jax.experimental import pallas as pl

NUM_EXPERTS = 16
TOP_K = 2
INPUT_DIM = 768
OUTPUT_DIM = 256
TOKEN_BLOCK = 1024


def _moe_block_kernel(x_ref, wg_ref, bg_ref, we_ref, be_ref, o_ref):
    x = x_ref[...]
    logits = jax.lax.dot_general(
        x, wg_ref[...], (((1,), (1,)), ((), ())),
        preferred_element_type=jnp.float32) + bg_ref[...]
    w = jax.nn.softmax(logits, axis=-1)
    e_iota = jax.lax.broadcasted_iota(jnp.int32, w.shape, 1)
    i1 = jnp.argmax(w, axis=-1)[:, None]
    v1 = jnp.max(w, axis=-1)[:, None]
    w2 = jnp.where(e_iota == i1, -jnp.inf, w)
    i2 = jnp.argmax(w2, axis=-1)[:, None]
    v2 = jnp.max(w2, axis=-1)[:, None]
    cw = (jnp.where(e_iota == i1, v1, 0.0)
          + jnp.where(e_iota == i2, v2, 0.0))

    xb = x.astype(jnp.bfloat16)
    w2d = we_ref[...].reshape(
        NUM_EXPERTS * OUTPUT_DIM, INPUT_DIM).astype(jnp.bfloat16)
    y_all = jax.lax.dot_general(
        xb, w2d, (((1,), (1,)), ((), ())),
        preferred_element_type=jnp.float32)
    acc = jnp.zeros((x.shape[0], OUTPUT_DIM), jnp.float32)
    for e in range(NUM_EXPERTS):
        ye = y_all[:, e * OUTPUT_DIM:(e + 1) * OUTPUT_DIM]
        acc = acc + cw[:, e][:, None] * (ye + be_ref[e][None, :])
    o_ref[...] = acc


@jax.jit
def kernel(x, Wg, bg, We, be):
    bs, L, d = x.shape
    n_tokens = bs * L
    xf = x.reshape(n_tokens, d)
    grid = (n_tokens // TOKEN_BLOCK,)
    out = pl.pallas_call(
        _moe_block_kernel,
        grid=grid,
        in_specs=[
            pl.BlockSpec((TOKEN_BLOCK, d), lambda i: (i, 0)),
            pl.BlockSpec((NUM_EXPERTS, d), lambda i: (0, 0)),
            pl.BlockSpec((1, NUM_EXPERTS), lambda i: (0, 0)),
            pl.BlockSpec((NUM_EXPERTS, OUTPUT_DIM, d), lambda i: (0, 0, 0)),
            pl.BlockSpec((NUM_EXPERTS, OUTPUT_DIM), lambda i: (0, 0)),
        ],
        out_specs=pl.BlockSpec((TOKEN_BLOCK, OUTPUT_DIM), lambda i: (i, 0)),
        out_shape=jax.ShapeDtypeStruct((n_tokens, OUTPUT_DIM), jnp.float32),
    )(xf, Wg, bg.reshape(1, NUM_EXPERTS), We, be)
    return out.reshape(bs, L, OUTPUT_DIM)
